# baseline (speedup 1.0000x reference)
import jax
import jax.numpy as jnp
from jax import lax
from jax.experimental import pallas as pl
from jax.experimental.pallas import tpu as pltpu
from jax.experimental.pallas import tpu_sc as plsc

N = 10000
NP = 10240
E = 160000
EP = 163840
H = 256
HH = 128
L = 4
G = 128
MAXD = 20
V2 = 5002
VP = 5120
NC = 2
NS = 16
BR = 1280
NRB = NP // BR
f32 = jnp.float32
i32 = jnp.int32


def _mesh():
    return plsc.VectorSubcoreMesh(core_axis_name="c", subcore_axis_name="s")


def _sc_embed_body(xa, aemb, attr_g, idxa, sa, sb, sem0, sem1):
    c = lax.axis_index("c")
    s = lax.axis_index("s")
    w = s * NC + c
    base = w * 320
    pltpu.sync_copy(xa.at[pl.ds(base, 320)], idxa)
    stgs = [sa, sb]
    sems = [sem0, sem1]
    d = pltpu.async_copy(aemb.at[idxa.at[pl.ds(0, 80)]], sa, sem0)
    for k in range(4):
        if k + 1 < 4:
            dn = pltpu.async_copy(
                aemb.at[idxa.at[pl.ds((k + 1) * 80, 80)]],
                stgs[(k + 1) % 2], sems[(k + 1) % 2])
        else:
            dn = None
        d.wait()
        pltpu.sync_copy(stgs[k % 2], attr_g.at[pl.ds(base + k * 80, 80)])
        d = dn


def _sc_embed(x1, aemb):
    return pl.kernel(
        _sc_embed_body,
        out_type=[jax.ShapeDtypeStruct((NP, H), f32)],
        mesh=_mesh(),
        scratch_types=[
            pltpu.VMEM((320,), i32),
            pltpu.VMEM((80, H), f32),
            pltpu.VMEM((80, H), f32),
            pltpu.SemaphoreType.DMA,
            pltpu.SemaphoreType.DMA,
        ],
    )(x1, aemb)[0]


def _sc_deg_body(zhbm, ohbm, dstr, out, zb, ob, dstb, accum):
    c = lax.axis_index("c")
    s = lax.axis_index("s")
    w = s * NC + c
    pltpu.sync_copy(zhbm, zb)
    pltpu.sync_copy(ohbm, ob)
    for k in range(5):
        pltpu.sync_copy(zb, accum.at[pl.ds(s * 640 + k * 128, 128)])
    plsc.subcore_barrier()
    pltpu.sync_copy(dstr.at[w], dstb)
    for g in range(40):
        pltpu.sync_copy(ob, accum.at[dstb.at[g]], add=True)
    plsc.subcore_barrier()
    pltpu.sync_copy(accum.at[pl.ds(s * 640, 640)],
                    out.at[c, pl.ds(s * 640, 640)])


def _sc_deg(zhbm, ohbm, dstr32):
    return pl.kernel(
        _sc_deg_body,
        out_type=[jax.ShapeDtypeStruct((NC, NP, HH), f32)],
        mesh=_mesh(),
        scratch_types=[
            pltpu.VMEM((128, HH), f32),
            pltpu.VMEM((128, HH), f32),
            pltpu.VMEM((40, 128), i32),
            pltpu.VMEM_SHARED((NP, HH), f32),
        ],
    )(zhbm, ohbm, dstr32)[0]


NCHUNK = 80
NGRP = 40


def _sc_agg_body(zhbm, hws, srcr, dstr, agg,
                 srcb, dstb, stg0, stg1, accum, gs0, gs1, ss0, ss1):
    c = lax.axis_index("c")
    s = lax.axis_index("s")
    stgs = [stg0, stg1]
    gsems = [gs0, gs1]
    ssems = [ss0, ss1]

    pltpu.sync_copy(zhbm, stg0)
    for k in range(5):
        pltpu.sync_copy(stg0, accum.at[pl.ds(s * 640 + k * 128, 128)])
    plsc.subcore_barrier()

    for grp in range(NCHUNK // NGRP):
        pltpu.sync_copy(srcr.at[c, s, pl.ds(grp * NGRP, NGRP)], srcb)
        pltpu.sync_copy(dstr.at[s, pl.ds(grp * NGRP, NGRP)], dstb)
        gd = [None, None]
        sd = [None, None]
        for g in range(NGRP):
            b = g % 2
            if sd[b] is not None:
                sd[b].wait()
            gd[b] = pltpu.async_copy(hws.at[srcb.at[g]], stgs[b], gsems[b])
            gd[b].wait()
            sd[b] = pltpu.async_copy(stgs[b], accum.at[dstb.at[g]],
                                     ssems[b], add=True)
        sd[0].wait()
        sd[1].wait()

    plsc.subcore_barrier()
    pltpu.sync_copy(accum.at[pl.ds(s * 640, 640)],
                    agg.at[c, pl.ds(s * 640, 640)])


def _sc_agg(zhbm, hws, srcr2, dstr):
    return pl.kernel(
        _sc_agg_body,
        out_type=[jax.ShapeDtypeStruct((NC, NP, HH), f32)],
        mesh=_mesh(),
        scratch_types=[
            pltpu.VMEM((NGRP, 128), i32),
            pltpu.VMEM((NGRP, 128), i32),
            pltpu.VMEM((128, HH), f32),
            pltpu.VMEM((128, HH), f32),
            pltpu.VMEM_SHARED((NP, HH), f32),
            pltpu.SemaphoreType.DMA,
            pltpu.SemaphoreType.DMA,
            pltpu.SemaphoreType.DMA,
            pltpu.SemaphoreType.DMA,
        ],
    )(zhbm, hws, srcr2, dstr)[0]


def _tc0_body(degw, ag, x0, dep, temb, demb, w0,
              h_out, dinv_out, hws_out):
    oh_t = (lax.broadcasted_iota(i32, (BR, 98), 1) == x0[...]).astype(f32)
    oh_d = (lax.broadcasted_iota(i32, (BR, MAXD + 1), 1)
            == dep[...]).astype(f32)
    h = (ag[...] + jnp.dot(oh_t, temb[...], preferred_element_type=f32)
         + jnp.dot(oh_d, demb[...], preferred_element_type=f32))
    deg = degw[0, :, 0] + degw[1, :, 0] + 1.0
    dinv = lax.rsqrt(jnp.maximum(deg, 1.0))[:, None]
    hw = jnp.dot(h, w0[...], preferred_element_type=f32) * dinv
    h_out[...] = h
    dinv_out[...] = dinv
    half = pl.program_id(0) < NRB
    hws_out[...] = jnp.where(half, hw[:, :HH], hw[:, HH:])


def _tc0(degw, ag, x0, dep, temb, demb, w0):
    return pl.pallas_call(
        _tc0_body,
        grid=(2 * NRB,),
        in_specs=[
            pl.BlockSpec((NC, BR, HH), lambda r: (0, r % NRB, 0)),
            pl.BlockSpec((BR, H), lambda r: (r % NRB, 0)),
            pl.BlockSpec((BR, 1), lambda r: (r % NRB, 0)),
            pl.BlockSpec((BR, 1), lambda r: (r % NRB, 0)),
            pl.BlockSpec((98, H), lambda r: (0, 0)),
            pl.BlockSpec((MAXD + 1, H), lambda r: (0, 0)),
            pl.BlockSpec((H, H), lambda r: (0, 0)),
        ],
        out_specs=[
            pl.BlockSpec((BR, H), lambda r: (r % NRB, 0)),
            pl.BlockSpec((BR, 1), lambda r: (r % NRB, 0)),
            pl.BlockSpec((BR, HH), lambda r: (r, 0)),
        ],
        out_shape=[
            jax.ShapeDtypeStruct((NP, H), f32),
            jax.ShapeDtypeStruct((NP, 1), f32),
            jax.ShapeDtypeStruct((2 * NP, HH), f32),
        ],
    )(degw, ag, x0, dep, temb, demb, w0)


def _tc_stats_body(agg, hwa, hwb, dinv, bvec, conv_out, psum, psumsq):
    r = pl.program_id(0)
    a = jnp.concatenate([agg[0], agg[1]], axis=1)
    hwp = jnp.concatenate([hwa[...], hwb[...]], axis=1)
    conv = (a + hwp) * dinv[...] + bvec[...]
    row = lax.broadcasted_iota(i32, (BR, 1), 0) + r * BR
    mask = (row < N).astype(f32)
    cm = conv * mask
    conv_out[...] = conv
    psum[0, 0] = jnp.sum(cm, axis=0)
    psumsq[0, 0] = jnp.sum(cm * cm, axis=0)


def _tc_stats(agg, hws, dinv, bvec):
    return pl.pallas_call(
        _tc_stats_body,
        grid=(NRB,),
        in_specs=[
            pl.BlockSpec((NC, BR, HH), lambda r: (0, r, 0)),
            pl.BlockSpec((BR, HH), lambda r: (r, 0)),
            pl.BlockSpec((BR, HH), lambda r: (r + NRB, 0)),
            pl.BlockSpec((BR, 1), lambda r: (r, 0)),
            pl.BlockSpec((1, H), lambda r: (0, 0)),
        ],
        out_specs=[
            pl.BlockSpec((BR, H), lambda r: (r, 0)),
            pl.BlockSpec((1, 1, H), lambda r: (r, 0, 0)),
            pl.BlockSpec((1, 1, H), lambda r: (r, 0, 0)),
        ],
        out_shape=[
            jax.ShapeDtypeStruct((NP, H), f32),
            jax.ShapeDtypeStruct((NRB, 1, H), f32),
            jax.ShapeDtypeStruct((NRB, 1, H), f32),
        ],
    )(agg, hws, hws, dinv, bvec)


def _tc_norm_mid_body(conv, psum, psumsq, hid, dinv, gam, bet, wn,
                      h_out, hws_out):
    mu = jnp.sum(psum[...], axis=0) / N
    var = jnp.sum(psumsq[...], axis=0) / N - mu * mu
    hn = (conv[...] - mu) * lax.rsqrt(var + 1e-5) * gam[...] + bet[...]
    hn = jnp.maximum(hn, 0.0) + hid[...]
    h_out[...] = hn
    nhw = jnp.dot(hn, wn[...], preferred_element_type=f32) * dinv[...]
    half = pl.program_id(0) < NRB
    hws_out[...] = jnp.where(half, nhw[:, :HH], nhw[:, HH:])


def _tc_norm_mid(conv, psum, psumsq, hid, dinv, gam, bet, wn):
    return pl.pallas_call(
        _tc_norm_mid_body,
        grid=(2 * NRB,),
        in_specs=[
            pl.BlockSpec((BR, H), lambda r: (r % NRB, 0)),
            pl.BlockSpec((NRB, 1, H), lambda r: (0, 0, 0)),
            pl.BlockSpec((NRB, 1, H), lambda r: (0, 0, 0)),
            pl.BlockSpec((BR, H), lambda r: (r % NRB, 0)),
            pl.BlockSpec((BR, 1), lambda r: (r % NRB, 0)),
            pl.BlockSpec((1, H), lambda r: (0, 0)),
            pl.BlockSpec((1, H), lambda r: (0, 0)),
            pl.BlockSpec((H, H), lambda r: (0, 0)),
        ],
        out_specs=[
            pl.BlockSpec((BR, H), lambda r: (r % NRB, 0)),
            pl.BlockSpec((BR, HH), lambda r: (r, 0)),
        ],
        out_shape=[
            jax.ShapeDtypeStruct((NP, H), f32),
            jax.ShapeDtypeStruct((2 * NP, HH), f32),
        ],
    )(conv, psum, psumsq, hid, dinv, gam, bet, wn)


def _tc_norm_last_body(conv, psum, psumsq, hid, gam, bet, h_out):
    mu = jnp.sum(psum[...], axis=0) / N
    var = jnp.sum(psumsq[...], axis=0) / N - mu * mu
    hn = (conv[...] - mu) * lax.rsqrt(var + 1e-5) * gam[...] + bet[...]
    h_out[...] = jnp.maximum(hn, 0.0) + hid[...]


def _tc_norm_last(conv, psum, psumsq, hid, gam, bet):
    return pl.pallas_call(
        _tc_norm_last_body,
        grid=(NRB,),
        in_specs=[
            pl.BlockSpec((BR, H), lambda r: (r, 0)),
            pl.BlockSpec((NRB, 1, H), lambda r: (0, 0, 0)),
            pl.BlockSpec((NRB, 1, H), lambda r: (0, 0, 0)),
            pl.BlockSpec((BR, H), lambda r: (r, 0)),
            pl.BlockSpec((1, H), lambda r: (0, 0)),
            pl.BlockSpec((1, H), lambda r: (0, 0)),
        ],
        out_specs=pl.BlockSpec((BR, H), lambda r: (r, 0)),
        out_shape=jax.ShapeDtypeStruct((NP, H), f32),
    )(conv, psum, psumsq, hid, gam, bet)


def _tc_pool_body(h, batch, wt, bt, out):
    oh = (lax.broadcasted_iota(i32, (G, NP), 0) == batch[...]).astype(f32)
    cnt = jnp.sum(oh, axis=1, keepdims=True)
    pooled = jnp.dot(oh, h[...], preferred_element_type=f32)
    pooled = pooled / jnp.maximum(cnt, 1.0)
    out[0] = jnp.dot(pooled, wt[0], preferred_element_type=f32) + bt[0]


def _tc_pool(h, batch2d, wt, bt):
    return pl.pallas_call(
        _tc_pool_body,
        grid=(5,),
        in_specs=[
            pl.BlockSpec((NP, H), lambda s: (0, 0)),
            pl.BlockSpec((1, NP), lambda s: (0, 0)),
            pl.BlockSpec((1, H, VP), lambda s: (s, 0, 0)),
            pl.BlockSpec((1, 1, VP), lambda s: (s, 0, 0)),
        ],
        out_specs=pl.BlockSpec((1, G, VP), lambda s: (s, 0, 0)),
        out_shape=jax.ShapeDtypeStruct((5, G, VP), f32),
    )(h, batch2d, wt, bt)


def kernel(x, edge_index, node_depth, batch, type_emb, attr_emb, depth_emb,
           Ws, bs, gammas, betas, Wt, bt):
    x0 = jnp.pad(x[:, 0], (0, NP - N))
    x1 = jnp.pad(x[:, 1], (0, NP - N))
    dep = jnp.pad(jnp.minimum(node_depth[:, 0], MAXD), (0, NP - N))
    pad_e = jnp.full((EP - E,), N, i32)
    src_p = jnp.concatenate([edge_index[0], pad_e])
    dst_p = jnp.concatenate([edge_index[1], pad_e])
    srcr2 = jnp.stack([src_p, src_p + NP]).reshape(NC, NS, NCHUNK, 128)
    dstr = dst_p.reshape(NS, NCHUNK, 128)
    dstr32 = dst_p.reshape(32, 40, 128)
    zconst = jnp.zeros((128, HH), f32)
    oconst = jnp.ones((128, HH), f32)

    ag = _sc_embed(x1, attr_emb)
    degw = _sc_deg(zconst, oconst, dstr32)
    h, dinv, hws = _tc0(degw, ag, x0.reshape(NP, 1),
                        dep.reshape(NP, 1), type_emb, depth_emb, Ws[0])
    for l in range(L):
        agg = _sc_agg(zconst, hws, srcr2, dstr)
        gam = gammas[l].reshape(1, H)
        bet = betas[l].reshape(1, H)
        bv = bs[l].reshape(1, H)
        conv, psum, psumsq = _tc_stats(agg, hws, dinv, bv)
        if l < L - 1:
            h, hws = _tc_norm_mid(conv, psum, psumsq, h, dinv, gam,
                                  bet, Ws[l + 1])
        else:
            h = _tc_norm_last(conv, psum, psumsq, h, gam, bet)

    batch_p = jnp.pad(batch, (0, NP - N), constant_values=2 * G)
    wt_p = jnp.pad(Wt, ((0, 0), (0, 0), (0, VP - V2)))
    bt_p = jnp.pad(bt, ((0, 0), (0, VP - V2))).reshape(5, 1, VP)
    preds = _tc_pool(h, batch_p.reshape(1, NP), wt_p, bt_p)
    return preds[:, :, :V2]


# speedup vs baseline: 1.0553x; 1.0553x over previous
import jax
import jax.numpy as jnp
from jax import lax
from jax.experimental import pallas as pl
from jax.experimental.pallas import tpu as pltpu
from jax.experimental.pallas import tpu_sc as plsc

N = 10000
NP = 10240
E = 160000
EP = 163840
H = 256
HH = 128
L = 4
G = 128
MAXD = 20
V2 = 5002
VP = 5120
NC = 2
NS = 16
BR = 1280
NRB = NP // BR
f32 = jnp.float32
i32 = jnp.int32


def _mesh():
    return plsc.VectorSubcoreMesh(core_axis_name="c", subcore_axis_name="s")


def _sc_embed_body(xa, aemb, attr_g, idxa, sa, sb, sem0, sem1):
    c = lax.axis_index("c")
    s = lax.axis_index("s")
    w = s * NC + c
    base = w * 320
    pltpu.sync_copy(xa.at[pl.ds(base, 320)], idxa)
    stgs = [sa, sb]
    sems = [sem0, sem1]
    d = pltpu.async_copy(aemb.at[idxa.at[pl.ds(0, 80)]], sa, sem0)
    for k in range(4):
        if k + 1 < 4:
            dn = pltpu.async_copy(
                aemb.at[idxa.at[pl.ds((k + 1) * 80, 80)]],
                stgs[(k + 1) % 2], sems[(k + 1) % 2])
        else:
            dn = None
        d.wait()
        pltpu.sync_copy(stgs[k % 2], attr_g.at[pl.ds(base + k * 80, 80)])
        d = dn


def _sc_embed(x1, aemb):
    return pl.kernel(
        _sc_embed_body,
        out_type=[jax.ShapeDtypeStruct((NP, H), f32)],
        mesh=_mesh(),
        scratch_types=[
            pltpu.VMEM((320,), i32),
            pltpu.VMEM((80, H), f32),
            pltpu.VMEM((80, H), f32),
            pltpu.SemaphoreType.DMA,
            pltpu.SemaphoreType.DMA,
        ],
    )(x1, aemb)[0]


def _sc_deg_body(zhbm, ohbm, dstr, out, zb, ob, dstb, accum):
    c = lax.axis_index("c")
    s = lax.axis_index("s")
    w = s * NC + c
    pltpu.sync_copy(zhbm, zb)
    pltpu.sync_copy(ohbm, ob)
    for k in range(5):
        pltpu.sync_copy(zb, accum.at[pl.ds(s * 640 + k * 128, 128)])
    plsc.subcore_barrier()
    pltpu.sync_copy(dstr.at[w], dstb)
    for g in range(40):
        pltpu.sync_copy(ob, accum.at[dstb.at[g]], add=True)
    plsc.subcore_barrier()
    pltpu.sync_copy(accum.at[pl.ds(s * 640, 640)],
                    out.at[c, pl.ds(s * 640, 640)])


def _sc_deg(zhbm, ohbm, dstr32):
    return pl.kernel(
        _sc_deg_body,
        out_type=[jax.ShapeDtypeStruct((NC, NP, HH), f32)],
        mesh=_mesh(),
        scratch_types=[
            pltpu.VMEM((128, HH), f32),
            pltpu.VMEM((128, HH), f32),
            pltpu.VMEM((40, 128), i32),
            pltpu.VMEM_SHARED((NP, HH), f32),
        ],
    )(zhbm, ohbm, dstr32)[0]


NCHUNK = 80
NGRP = 40


def _sc_agg_body(zhbm, hws, srcr, dstr, agg,
                 srcb, dstb, stg0, stg1, accum, gs0, gs1, ss0, ss1):
    c = lax.axis_index("c")
    s = lax.axis_index("s")
    stgs = [stg0, stg1]
    gsems = [gs0, gs1]
    ssems = [ss0, ss1]

    pltpu.sync_copy(zhbm, stg0)
    for k in range(5):
        pltpu.sync_copy(stg0, accum.at[pl.ds(s * 640 + k * 128, 128)])
    plsc.subcore_barrier()

    for grp in range(NCHUNK // NGRP):
        pltpu.sync_copy(srcr.at[c, s, pl.ds(grp * NGRP, NGRP)], srcb)
        pltpu.sync_copy(dstr.at[s, pl.ds(grp * NGRP, NGRP)], dstb)
        gd = [None, None]
        sd = [None, None]
        gd[0] = pltpu.async_copy(hws.at[srcb.at[0]], stgs[0], gsems[0])
        for g in range(NGRP):
            b = g % 2
            b2 = 1 - b
            if g + 1 < NGRP:
                if sd[b2] is not None:
                    sd[b2].wait()
                gd[b2] = pltpu.async_copy(hws.at[srcb.at[g + 1]],
                                          stgs[b2], gsems[b2])
            gd[b].wait()
            sd[b] = pltpu.async_copy(stgs[b], accum.at[dstb.at[g]],
                                     ssems[b], add=True)
        sd[0].wait()
        sd[1].wait()

    plsc.subcore_barrier()
    pltpu.sync_copy(accum.at[pl.ds(s * 640, 640)],
                    agg.at[c, pl.ds(s * 640, 640)])


def _sc_agg(zhbm, hws, srcr2, dstr):
    return pl.kernel(
        _sc_agg_body,
        out_type=[jax.ShapeDtypeStruct((NC, NP, HH), f32)],
        mesh=_mesh(),
        scratch_types=[
            pltpu.VMEM((NGRP, 128), i32),
            pltpu.VMEM((NGRP, 128), i32),
            pltpu.VMEM((128, HH), f32),
            pltpu.VMEM((128, HH), f32),
            pltpu.VMEM_SHARED((NP, HH), f32),
            pltpu.SemaphoreType.DMA,
            pltpu.SemaphoreType.DMA,
            pltpu.SemaphoreType.DMA,
            pltpu.SemaphoreType.DMA,
        ],
    )(zhbm, hws, srcr2, dstr)[0]


def _tc0_body(degw, ag, x0, dep, temb, demb, w0,
              h_out, dinv_out, hws_out):
    oh_t = (lax.broadcasted_iota(i32, (BR, 98), 1) == x0[...]).astype(f32)
    oh_d = (lax.broadcasted_iota(i32, (BR, MAXD + 1), 1)
            == dep[...]).astype(f32)
    h = (ag[...] + jnp.dot(oh_t, temb[...], preferred_element_type=f32)
         + jnp.dot(oh_d, demb[...], preferred_element_type=f32))
    deg = degw[0, :, 0] + degw[1, :, 0] + 1.0
    dinv = lax.rsqrt(jnp.maximum(deg, 1.0))[:, None]
    hw = jnp.dot(h, w0[...], preferred_element_type=f32) * dinv
    h_out[...] = h
    dinv_out[...] = dinv
    half = pl.program_id(0) < NRB
    hws_out[...] = jnp.where(half, hw[:, :HH], hw[:, HH:])


def _tc0(degw, ag, x0, dep, temb, demb, w0):
    return pl.pallas_call(
        _tc0_body,
        grid=(2 * NRB,),
        in_specs=[
            pl.BlockSpec((NC, BR, HH), lambda r: (0, r % NRB, 0)),
            pl.BlockSpec((BR, H), lambda r: (r % NRB, 0)),
            pl.BlockSpec((BR, 1), lambda r: (r % NRB, 0)),
            pl.BlockSpec((BR, 1), lambda r: (r % NRB, 0)),
            pl.BlockSpec((98, H), lambda r: (0, 0)),
            pl.BlockSpec((MAXD + 1, H), lambda r: (0, 0)),
            pl.BlockSpec((H, H), lambda r: (0, 0)),
        ],
        out_specs=[
            pl.BlockSpec((BR, H), lambda r: (r % NRB, 0)),
            pl.BlockSpec((BR, 1), lambda r: (r % NRB, 0)),
            pl.BlockSpec((BR, HH), lambda r: (r, 0)),
        ],
        out_shape=[
            jax.ShapeDtypeStruct((NP, H), f32),
            jax.ShapeDtypeStruct((NP, 1), f32),
            jax.ShapeDtypeStruct((2 * NP, HH), f32),
        ],
    )(degw, ag, x0, dep, temb, demb, w0)


def _tc_stats_body(agg, hwa, hwb, dinv, bvec, conv_out, psum, psumsq):
    r = pl.program_id(0)
    a = jnp.concatenate([agg[0], agg[1]], axis=1)
    hwp = jnp.concatenate([hwa[...], hwb[...]], axis=1)
    conv = (a + hwp) * dinv[...] + bvec[...]
    row = lax.broadcasted_iota(i32, (BR, 1), 0) + r * BR
    mask = (row < N).astype(f32)
    cm = conv * mask
    conv_out[...] = conv
    psum[0, 0] = jnp.sum(cm, axis=0)
    psumsq[0, 0] = jnp.sum(cm * cm, axis=0)


def _tc_stats(agg, hws, dinv, bvec):
    return pl.pallas_call(
        _tc_stats_body,
        grid=(NRB,),
        in_specs=[
            pl.BlockSpec((NC, BR, HH), lambda r: (0, r, 0)),
            pl.BlockSpec((BR, HH), lambda r: (r, 0)),
            pl.BlockSpec((BR, HH), lambda r: (r + NRB, 0)),
            pl.BlockSpec((BR, 1), lambda r: (r, 0)),
            pl.BlockSpec((1, H), lambda r: (0, 0)),
        ],
        out_specs=[
            pl.BlockSpec((BR, H), lambda r: (r, 0)),
            pl.BlockSpec((1, 1, H), lambda r: (r, 0, 0)),
            pl.BlockSpec((1, 1, H), lambda r: (r, 0, 0)),
        ],
        out_shape=[
            jax.ShapeDtypeStruct((NP, H), f32),
            jax.ShapeDtypeStruct((NRB, 1, H), f32),
            jax.ShapeDtypeStruct((NRB, 1, H), f32),
        ],
    )(agg, hws, hws, dinv, bvec)


def _tc_norm_mid_body(conv, psum, psumsq, hid, dinv, gam, bet, wn,
                      h_out, hws_out):
    mu = jnp.sum(psum[...], axis=0) / N
    var = jnp.sum(psumsq[...], axis=0) / N - mu * mu
    hn = (conv[...] - mu) * lax.rsqrt(var + 1e-5) * gam[...] + bet[...]
    hn = jnp.maximum(hn, 0.0) + hid[...]
    h_out[...] = hn
    nhw = jnp.dot(hn, wn[...], preferred_element_type=f32) * dinv[...]
    half = pl.program_id(0) < NRB
    hws_out[...] = jnp.where(half, nhw[:, :HH], nhw[:, HH:])


def _tc_norm_mid(conv, psum, psumsq, hid, dinv, gam, bet, wn):
    return pl.pallas_call(
        _tc_norm_mid_body,
        grid=(2 * NRB,),
        in_specs=[
            pl.BlockSpec((BR, H), lambda r: (r % NRB, 0)),
            pl.BlockSpec((NRB, 1, H), lambda r: (0, 0, 0)),
            pl.BlockSpec((NRB, 1, H), lambda r: (0, 0, 0)),
            pl.BlockSpec((BR, H), lambda r: (r % NRB, 0)),
            pl.BlockSpec((BR, 1), lambda r: (r % NRB, 0)),
            pl.BlockSpec((1, H), lambda r: (0, 0)),
            pl.BlockSpec((1, H), lambda r: (0, 0)),
            pl.BlockSpec((H, H), lambda r: (0, 0)),
        ],
        out_specs=[
            pl.BlockSpec((BR, H), lambda r: (r % NRB, 0)),
            pl.BlockSpec((BR, HH), lambda r: (r, 0)),
        ],
        out_shape=[
            jax.ShapeDtypeStruct((NP, H), f32),
            jax.ShapeDtypeStruct((2 * NP, HH), f32),
        ],
    )(conv, psum, psumsq, hid, dinv, gam, bet, wn)


def _tc_norm_last_body(conv, psum, psumsq, hid, gam, bet, h_out):
    mu = jnp.sum(psum[...], axis=0) / N
    var = jnp.sum(psumsq[...], axis=0) / N - mu * mu
    hn = (conv[...] - mu) * lax.rsqrt(var + 1e-5) * gam[...] + bet[...]
    h_out[...] = jnp.maximum(hn, 0.0) + hid[...]


def _tc_norm_last(conv, psum, psumsq, hid, gam, bet):
    return pl.pallas_call(
        _tc_norm_last_body,
        grid=(NRB,),
        in_specs=[
            pl.BlockSpec((BR, H), lambda r: (r, 0)),
            pl.BlockSpec((NRB, 1, H), lambda r: (0, 0, 0)),
            pl.BlockSpec((NRB, 1, H), lambda r: (0, 0, 0)),
            pl.BlockSpec((BR, H), lambda r: (r, 0)),
            pl.BlockSpec((1, H), lambda r: (0, 0)),
            pl.BlockSpec((1, H), lambda r: (0, 0)),
        ],
        out_specs=pl.BlockSpec((BR, H), lambda r: (r, 0)),
        out_shape=jax.ShapeDtypeStruct((NP, H), f32),
    )(conv, psum, psumsq, hid, gam, bet)


def _tc_pool_body(h, batch, wt, bt, out):
    oh = (lax.broadcasted_iota(i32, (G, NP), 0) == batch[...]).astype(f32)
    cnt = jnp.sum(oh, axis=1, keepdims=True)
    pooled = jnp.dot(oh, h[...], preferred_element_type=f32)
    pooled = pooled / jnp.maximum(cnt, 1.0)
    out[0] = jnp.dot(pooled, wt[0], preferred_element_type=f32) + bt[0]


def _tc_pool(h, batch2d, wt, bt):
    return pl.pallas_call(
        _tc_pool_body,
        grid=(5,),
        in_specs=[
            pl.BlockSpec((NP, H), lambda s: (0, 0)),
            pl.BlockSpec((1, NP), lambda s: (0, 0)),
            pl.BlockSpec((1, H, VP), lambda s: (s, 0, 0)),
            pl.BlockSpec((1, 1, VP), lambda s: (s, 0, 0)),
        ],
        out_specs=pl.BlockSpec((1, G, VP), lambda s: (s, 0, 0)),
        out_shape=jax.ShapeDtypeStruct((5, G, VP), f32),
    )(h, batch2d, wt, bt)


def kernel(x, edge_index, node_depth, batch, type_emb, attr_emb, depth_emb,
           Ws, bs, gammas, betas, Wt, bt):
    x0 = jnp.pad(x[:, 0], (0, NP - N))
    x1 = jnp.pad(x[:, 1], (0, NP - N))
    dep = jnp.pad(jnp.minimum(node_depth[:, 0], MAXD), (0, NP - N))
    pad_e = jnp.full((EP - E,), N, i32)
    src_p = jnp.concatenate([edge_index[0], pad_e])
    dst_p = jnp.concatenate([edge_index[1], pad_e])
    srcr2 = jnp.stack([src_p, src_p + NP]).reshape(NC, NS, NCHUNK, 128)
    dstr = dst_p.reshape(NS, NCHUNK, 128)
    dstr32 = dst_p.reshape(32, 40, 128)
    zconst = jnp.zeros((128, HH), f32)
    oconst = jnp.ones((128, HH), f32)

    ag = _sc_embed(x1, attr_emb)
    degw = _sc_deg(zconst, oconst, dstr32)
    h, dinv, hws = _tc0(degw, ag, x0.reshape(NP, 1),
                        dep.reshape(NP, 1), type_emb, depth_emb, Ws[0])
    for l in range(L):
        agg = _sc_agg(zconst, hws, srcr2, dstr)
        gam = gammas[l].reshape(1, H)
        bet = betas[l].reshape(1, H)
        bv = bs[l].reshape(1, H)
        conv, psum, psumsq = _tc_stats(agg, hws, dinv, bv)
        if l < L - 1:
            h, hws = _tc_norm_mid(conv, psum, psumsq, h, dinv, gam,
                                  bet, Ws[l + 1])
        else:
            h = _tc_norm_last(conv, psum, psumsq, h, gam, bet)

    batch_p = jnp.pad(batch, (0, NP - N), constant_values=2 * G)
    wt_p = jnp.pad(Wt, ((0, 0), (0, 0), (0, VP - V2)))
    bt_p = jnp.pad(bt, ((0, 0), (0, VP - V2))).reshape(5, 1, VP)
    preds = _tc_pool(h, batch_p.reshape(1, NP), wt_p, bt_p)
    return preds[:, :, :V2]


# speedup vs baseline: 1.0780x; 1.0216x over previous
import jax
import jax.numpy as jnp
from jax import lax
from jax.experimental import pallas as pl
from jax.experimental.pallas import tpu as pltpu
from jax.experimental.pallas import tpu_sc as plsc

N = 10000
NP = 10240
E = 160000
EP = 163840
H = 256
HH = 128
L = 4
G = 128
MAXD = 20
V2 = 5002
VP = 5120
NC = 2
NS = 16
BR = 1280
NRB = NP // BR
f32 = jnp.float32
i32 = jnp.int32


def _mesh():
    return plsc.VectorSubcoreMesh(core_axis_name="c", subcore_axis_name="s")


def _sc_embed_body(xa, aemb, attr_g, idxa, sa, sb, sem0, sem1):
    c = lax.axis_index("c")
    s = lax.axis_index("s")
    w = s * NC + c
    base = w * 320
    pltpu.sync_copy(xa.at[pl.ds(base, 320)], idxa)
    stgs = [sa, sb]
    sems = [sem0, sem1]
    d = pltpu.async_copy(aemb.at[idxa.at[pl.ds(0, 80)]], sa, sem0)
    for k in range(4):
        if k + 1 < 4:
            dn = pltpu.async_copy(
                aemb.at[idxa.at[pl.ds((k + 1) * 80, 80)]],
                stgs[(k + 1) % 2], sems[(k + 1) % 2])
        else:
            dn = None
        d.wait()
        pltpu.sync_copy(stgs[k % 2], attr_g.at[pl.ds(base + k * 80, 80)])
        d = dn


def _sc_embed(x1, aemb):
    return pl.kernel(
        _sc_embed_body,
        out_type=[jax.ShapeDtypeStruct((NP, H), f32)],
        mesh=_mesh(),
        scratch_types=[
            pltpu.VMEM((320,), i32),
            pltpu.VMEM((80, H), f32),
            pltpu.VMEM((80, H), f32),
            pltpu.SemaphoreType.DMA,
            pltpu.SemaphoreType.DMA,
        ],
    )(x1, aemb)[0]


def _sc_deg_body(zhbm, ohbm, dstr, out, zb, ob, dstb, accum):
    c = lax.axis_index("c")
    s = lax.axis_index("s")
    w = s * NC + c
    pltpu.sync_copy(zhbm, zb)
    pltpu.sync_copy(ohbm, ob)
    for k in range(5):
        pltpu.sync_copy(zb, accum.at[pl.ds(s * 640 + k * 128, 128)])
    plsc.subcore_barrier()
    pltpu.sync_copy(dstr.at[w], dstb)
    for g in range(40):
        pltpu.sync_copy(ob, accum.at[dstb.at[g]], add=True)
    plsc.subcore_barrier()
    pltpu.sync_copy(accum.at[pl.ds(s * 640, 640)],
                    out.at[c, pl.ds(s * 640, 640)])


def _sc_deg(zhbm, ohbm, dstr32):
    return pl.kernel(
        _sc_deg_body,
        out_type=[jax.ShapeDtypeStruct((NC, NP, HH), f32)],
        mesh=_mesh(),
        scratch_types=[
            pltpu.VMEM((128, HH), f32),
            pltpu.VMEM((128, HH), f32),
            pltpu.VMEM((40, 128), i32),
            pltpu.VMEM_SHARED((NP, HH), f32),
        ],
    )(zhbm, ohbm, dstr32)[0]


NCHUNK = 80
NGRP = 40


def _sc_agg_body(zhbm, hws, srcr, dstr, agg,
                 srcb, dstb, stg0, stg1, accum, gs0, gs1, ss0, ss1):
    c = lax.axis_index("c")
    s = lax.axis_index("s")
    stgs = [stg0, stg1]
    gsems = [gs0, gs1]
    ssems = [ss0, ss1]

    pltpu.sync_copy(zhbm, stg0)
    for k in range(5):
        pltpu.sync_copy(stg0, accum.at[pl.ds(s * 640 + k * 128, 128)])
    plsc.subcore_barrier()

    for grp in range(NCHUNK // NGRP):
        pltpu.sync_copy(srcr.at[c, s, pl.ds(grp * NGRP, NGRP)], srcb)
        pltpu.sync_copy(dstr.at[s, pl.ds(grp * NGRP, NGRP)], dstb)
        gd = [None, None]
        sd = [None, None]
        gd[0] = pltpu.async_copy(hws.at[srcb.at[0]], stgs[0], gsems[0])
        for g in range(NGRP):
            b = g % 2
            b2 = 1 - b
            if g + 1 < NGRP:
                if sd[b2] is not None:
                    sd[b2].wait()
                gd[b2] = pltpu.async_copy(hws.at[srcb.at[g + 1]],
                                          stgs[b2], gsems[b2])
            gd[b].wait()
            sd[b] = pltpu.async_copy(stgs[b], accum.at[dstb.at[g]],
                                     ssems[b], add=True)
        sd[0].wait()
        sd[1].wait()

    plsc.subcore_barrier()
    pltpu.sync_copy(accum.at[pl.ds(s * 640, 640)],
                    agg.at[c, pl.ds(s * 640, 640)])


def _sc_agg(zhbm, hws, srcr2, dstr):
    return pl.kernel(
        _sc_agg_body,
        out_type=[jax.ShapeDtypeStruct((NC, NP, HH), f32)],
        mesh=_mesh(),
        scratch_types=[
            pltpu.VMEM((NGRP, 128), i32),
            pltpu.VMEM((NGRP, 128), i32),
            pltpu.VMEM((128, HH), f32),
            pltpu.VMEM((128, HH), f32),
            pltpu.VMEM_SHARED((NP, HH), f32),
            pltpu.SemaphoreType.DMA,
            pltpu.SemaphoreType.DMA,
            pltpu.SemaphoreType.DMA,
            pltpu.SemaphoreType.DMA,
        ],
    )(zhbm, hws, srcr2, dstr)[0]


def _tc0_body(degw, ag, x0, dep, temb, demb, w0,
              h_out, dinv_out, hw0_out, hw1_out):
    oh_t = (lax.broadcasted_iota(i32, (BR, 98), 1) == x0[...]).astype(f32)
    oh_d = (lax.broadcasted_iota(i32, (BR, MAXD + 1), 1)
            == dep[...]).astype(f32)
    h = (ag[...] + jnp.dot(oh_t, temb[...], preferred_element_type=f32)
         + jnp.dot(oh_d, demb[...], preferred_element_type=f32))
    deg = degw[0, :, 0] + degw[1, :, 0] + 1.0
    dinv = lax.rsqrt(jnp.maximum(deg, 1.0))[:, None]
    hw = jnp.dot(h, w0[...], preferred_element_type=f32) * dinv
    h_out[...] = h
    dinv_out[...] = dinv
    hw0_out[...] = hw[:, :HH]
    hw1_out[...] = hw[:, HH:]


def _tc0(degw, ag, x0, dep, temb, demb, w0):
    return pl.pallas_call(
        _tc0_body,
        grid=(NRB,),
        in_specs=[
            pl.BlockSpec((NC, BR, HH), lambda r: (0, r, 0)),
            pl.BlockSpec((BR, H), lambda r: (r, 0)),
            pl.BlockSpec((BR, 1), lambda r: (r, 0)),
            pl.BlockSpec((BR, 1), lambda r: (r, 0)),
            pl.BlockSpec((98, H), lambda r: (0, 0)),
            pl.BlockSpec((MAXD + 1, H), lambda r: (0, 0)),
            pl.BlockSpec((H, H), lambda r: (0, 0)),
        ],
        out_specs=[
            pl.BlockSpec((BR, H), lambda r: (r, 0)),
            pl.BlockSpec((BR, 1), lambda r: (r, 0)),
            pl.BlockSpec((BR, HH), lambda r: (r, 0)),
            pl.BlockSpec((BR, HH), lambda r: (r, 0)),
        ],
        out_shape=[
            jax.ShapeDtypeStruct((NP, H), f32),
            jax.ShapeDtypeStruct((NP, 1), f32),
            jax.ShapeDtypeStruct((NP, HH), f32),
            jax.ShapeDtypeStruct((NP, HH), f32),
        ],
    )(degw, ag, x0, dep, temb, demb, w0)


def _tc_stats_body(agg, hwa, hwb, dinv, bvec, conv_out, psum, psumsq):
    r = pl.program_id(0)
    a = jnp.concatenate([agg[0], agg[1]], axis=1)
    hwp = jnp.concatenate([hwa[...], hwb[...]], axis=1)
    conv = (a + hwp) * dinv[...] + bvec[...]
    row = lax.broadcasted_iota(i32, (BR, 1), 0) + r * BR
    mask = (row < N).astype(f32)
    cm = conv * mask
    conv_out[...] = conv
    psum[0, 0] = jnp.sum(cm, axis=0)
    psumsq[0, 0] = jnp.sum(cm * cm, axis=0)


def _tc_stats(agg, hw0, hw1, dinv, bvec):
    return pl.pallas_call(
        _tc_stats_body,
        grid=(NRB,),
        in_specs=[
            pl.BlockSpec((NC, BR, HH), lambda r: (0, r, 0)),
            pl.BlockSpec((BR, HH), lambda r: (r, 0)),
            pl.BlockSpec((BR, HH), lambda r: (r, 0)),
            pl.BlockSpec((BR, 1), lambda r: (r, 0)),
            pl.BlockSpec((1, H), lambda r: (0, 0)),
        ],
        out_specs=[
            pl.BlockSpec((BR, H), lambda r: (r, 0)),
            pl.BlockSpec((1, 1, H), lambda r: (r, 0, 0)),
            pl.BlockSpec((1, 1, H), lambda r: (r, 0, 0)),
        ],
        out_shape=[
            jax.ShapeDtypeStruct((NP, H), f32),
            jax.ShapeDtypeStruct((NRB, 1, H), f32),
            jax.ShapeDtypeStruct((NRB, 1, H), f32),
        ],
    )(agg, hw0, hw1, dinv, bvec)


def _tc_norm_mid_body(conv, psum, psumsq, hid, dinv, gam, bet, wn,
                      h_out, hw0_out, hw1_out):
    mu = jnp.sum(psum[...], axis=0) / N
    var = jnp.sum(psumsq[...], axis=0) / N - mu * mu
    hn = (conv[...] - mu) * lax.rsqrt(var + 1e-5) * gam[...] + bet[...]
    hn = jnp.maximum(hn, 0.0) + hid[...]
    h_out[...] = hn
    nhw = jnp.dot(hn, wn[...], preferred_element_type=f32) * dinv[...]
    hw0_out[...] = nhw[:, :HH]
    hw1_out[...] = nhw[:, HH:]


def _tc_norm_mid(conv, psum, psumsq, hid, dinv, gam, bet, wn):
    return pl.pallas_call(
        _tc_norm_mid_body,
        grid=(NRB,),
        in_specs=[
            pl.BlockSpec((BR, H), lambda r: (r, 0)),
            pl.BlockSpec((NRB, 1, H), lambda r: (0, 0, 0)),
            pl.BlockSpec((NRB, 1, H), lambda r: (0, 0, 0)),
            pl.BlockSpec((BR, H), lambda r: (r, 0)),
            pl.BlockSpec((BR, 1), lambda r: (r, 0)),
            pl.BlockSpec((1, H), lambda r: (0, 0)),
            pl.BlockSpec((1, H), lambda r: (0, 0)),
            pl.BlockSpec((H, H), lambda r: (0, 0)),
        ],
        out_specs=[
            pl.BlockSpec((BR, H), lambda r: (r, 0)),
            pl.BlockSpec((BR, HH), lambda r: (r, 0)),
            pl.BlockSpec((BR, HH), lambda r: (r, 0)),
        ],
        out_shape=[
            jax.ShapeDtypeStruct((NP, H), f32),
            jax.ShapeDtypeStruct((NP, HH), f32),
            jax.ShapeDtypeStruct((NP, HH), f32),
        ],
    )(conv, psum, psumsq, hid, dinv, gam, bet, wn)


def _tc_norm_last_body(conv, psum, psumsq, hid, gam, bet, h_out):
    mu = jnp.sum(psum[...], axis=0) / N
    var = jnp.sum(psumsq[...], axis=0) / N - mu * mu
    hn = (conv[...] - mu) * lax.rsqrt(var + 1e-5) * gam[...] + bet[...]
    h_out[...] = jnp.maximum(hn, 0.0) + hid[...]


def _tc_norm_last(conv, psum, psumsq, hid, gam, bet):
    return pl.pallas_call(
        _tc_norm_last_body,
        grid=(NRB,),
        in_specs=[
            pl.BlockSpec((BR, H), lambda r: (r, 0)),
            pl.BlockSpec((NRB, 1, H), lambda r: (0, 0, 0)),
            pl.BlockSpec((NRB, 1, H), lambda r: (0, 0, 0)),
            pl.BlockSpec((BR, H), lambda r: (r, 0)),
            pl.BlockSpec((1, H), lambda r: (0, 0)),
            pl.BlockSpec((1, H), lambda r: (0, 0)),
        ],
        out_specs=pl.BlockSpec((BR, H), lambda r: (r, 0)),
        out_shape=jax.ShapeDtypeStruct((NP, H), f32),
    )(conv, psum, psumsq, hid, gam, bet)


def _tc_pool_body(h, batch, wt, bt, out):
    oh = (lax.broadcasted_iota(i32, (G, NP), 0) == batch[...]).astype(f32)
    cnt = jnp.sum(oh, axis=1, keepdims=True)
    pooled = jnp.dot(oh, h[...], preferred_element_type=f32)
    pooled = pooled / jnp.maximum(cnt, 1.0)
    out[0] = jnp.dot(pooled, wt[0], preferred_element_type=f32) + bt[0]


def _tc_pool(h, batch2d, wt, bt):
    return pl.pallas_call(
        _tc_pool_body,
        grid=(5,),
        in_specs=[
            pl.BlockSpec((NP, H), lambda s: (0, 0)),
            pl.BlockSpec((1, NP), lambda s: (0, 0)),
            pl.BlockSpec((1, H, VP), lambda s: (s, 0, 0)),
            pl.BlockSpec((1, 1, VP), lambda s: (s, 0, 0)),
        ],
        out_specs=pl.BlockSpec((1, G, VP), lambda s: (s, 0, 0)),
        out_shape=jax.ShapeDtypeStruct((5, G, VP), f32),
    )(h, batch2d, wt, bt)


def kernel(x, edge_index, node_depth, batch, type_emb, attr_emb, depth_emb,
           Ws, bs, gammas, betas, Wt, bt):
    x0 = jnp.pad(x[:, 0], (0, NP - N))
    x1 = jnp.pad(x[:, 1], (0, NP - N))
    dep = jnp.pad(jnp.minimum(node_depth[:, 0], MAXD), (0, NP - N))
    pad_e = jnp.full((EP - E,), N, i32)
    src_p = jnp.concatenate([edge_index[0], pad_e])
    dst_p = jnp.concatenate([edge_index[1], pad_e])
    srcr2 = jnp.stack([src_p, src_p + NP]).reshape(NC, NS, NCHUNK, 128)
    dstr = dst_p.reshape(NS, NCHUNK, 128)
    dstr32 = dst_p.reshape(32, 40, 128)
    zconst = jnp.zeros((128, HH), f32)
    oconst = jnp.ones((128, HH), f32)

    ag = _sc_embed(x1, attr_emb)
    degw = _sc_deg(zconst, oconst, dstr32)
    h, dinv, hw0, hw1 = _tc0(degw, ag, x0.reshape(NP, 1),
                             dep.reshape(NP, 1), type_emb, depth_emb,
                             Ws[0])
    for l in range(L):
        hws = jnp.concatenate([hw0, hw1], axis=0)
        agg = _sc_agg(zconst, hws, srcr2, dstr)
        gam = gammas[l].reshape(1, H)
        bet = betas[l].reshape(1, H)
        bv = bs[l].reshape(1, H)
        conv, psum, psumsq = _tc_stats(agg, hw0, hw1, dinv, bv)
        if l < L - 1:
            h, hw0, hw1 = _tc_norm_mid(conv, psum, psumsq, h, dinv, gam,
                                       bet, Ws[l + 1])
        else:
            h = _tc_norm_last(conv, psum, psumsq, h, gam, bet)

    batch_p = jnp.pad(batch, (0, NP - N), constant_values=2 * G)
    wt_p = jnp.pad(Wt, ((0, 0), (0, 0), (0, VP - V2)))
    bt_p = jnp.pad(bt, ((0, 0), (0, VP - V2))).reshape(5, 1, VP)
    preds = _tc_pool(h, batch_p.reshape(1, NP), wt_p, bt_p)
    return preds[:, :, :V2]


# speedup vs baseline: 1.1260x; 1.0445x over previous
import jax
import jax.numpy as jnp
from jax import lax
from jax.experimental import pallas as pl
from jax.experimental.pallas import tpu as pltpu
from jax.experimental.pallas import tpu_sc as plsc

N = 10000
NP = 10240
E = 160000
EP = 163840
H = 256
HH = 128
L = 4
G = 128
MAXD = 20
V2 = 5002
VP = 5120
NC = 2
NS = 16
BR = 1280
NRB = NP // BR
f32 = jnp.float32
i32 = jnp.int32


def _mesh():
    return plsc.VectorSubcoreMesh(core_axis_name="c", subcore_axis_name="s")


def _sc_embed_body(xa, aemb, attr_g, idxa, sa, sb, sem0, sem1):
    c = lax.axis_index("c")
    s = lax.axis_index("s")
    w = s * NC + c
    base = w * 320
    pltpu.sync_copy(xa.at[pl.ds(base, 320)], idxa)
    stgs = [sa, sb]
    sems = [sem0, sem1]
    d = pltpu.async_copy(aemb.at[idxa.at[pl.ds(0, 80)]], sa, sem0)
    for k in range(4):
        if k + 1 < 4:
            dn = pltpu.async_copy(
                aemb.at[idxa.at[pl.ds((k + 1) * 80, 80)]],
                stgs[(k + 1) % 2], sems[(k + 1) % 2])
        else:
            dn = None
        d.wait()
        pltpu.sync_copy(stgs[k % 2], attr_g.at[pl.ds(base + k * 80, 80)])
        d = dn


def _sc_embed(x1, aemb):
    return pl.kernel(
        _sc_embed_body,
        out_type=[jax.ShapeDtypeStruct((NP, H), f32)],
        mesh=_mesh(),
        scratch_types=[
            pltpu.VMEM((320,), i32),
            pltpu.VMEM((80, H), f32),
            pltpu.VMEM((80, H), f32),
            pltpu.SemaphoreType.DMA,
            pltpu.SemaphoreType.DMA,
        ],
    )(x1, aemb)[0]


def _sc_deg_body(zhbm, ohbm, dstr, out, zb, ob, dstb, accum):
    c = lax.axis_index("c")
    s = lax.axis_index("s")
    w = s * NC + c
    pltpu.sync_copy(zhbm, zb)
    pltpu.sync_copy(ohbm, ob)
    for k in range(5):
        pltpu.sync_copy(zb, accum.at[pl.ds(s * 640 + k * 128, 128)])
    plsc.subcore_barrier()
    pltpu.sync_copy(dstr.at[w], dstb)
    for g in range(40):
        pltpu.sync_copy(ob, accum.at[dstb.at[g]], add=True)
    plsc.subcore_barrier()
    pltpu.sync_copy(accum.at[pl.ds(s * 640, 640)],
                    out.at[c, pl.ds(s * 640, 640)])


def _sc_deg(zhbm, ohbm, dstr32):
    return pl.kernel(
        _sc_deg_body,
        out_type=[jax.ShapeDtypeStruct((NC, NP, HH), f32)],
        mesh=_mesh(),
        scratch_types=[
            pltpu.VMEM((128, HH), f32),
            pltpu.VMEM((128, HH), f32),
            pltpu.VMEM((40, 128), i32),
            pltpu.VMEM_SHARED((NP, HH), f32),
        ],
    )(zhbm, ohbm, dstr32)[0]


NCHUNK = 80
NGRP = 40


def _sc_agg_body(zhbm, hw0, hw1, srcr, dstr, agg,
                 srcb, dstb, stg0, stg1, accum, sem0, sem1):
    c = lax.axis_index("c")
    s = lax.axis_index("s")
    stgs = [stg0, stg1]
    sems = [sem0, sem1]

    pltpu.sync_copy(zhbm, stg0)
    for k in range(5):
        pltpu.sync_copy(stg0, accum.at[pl.ds(s * 640 + k * 128, 128)])
    plsc.subcore_barrier()

    def pipeline(hw):
        for grp in range(NCHUNK // NGRP):
            pltpu.sync_copy(srcr.at[s, pl.ds(grp * NGRP, NGRP)], srcb)
            pltpu.sync_copy(dstr.at[s, pl.ds(grp * NGRP, NGRP)], dstb)
            d = pltpu.async_copy(hw.at[srcb.at[0]], stgs[0], sems[0])
            for g in range(NGRP):
                b = g % 2
                if g + 1 < NGRP:
                    dn = pltpu.async_copy(
                        hw.at[srcb.at[g + 1]], stgs[1 - b], sems[1 - b])
                else:
                    dn = None
                d.wait()
                pltpu.sync_copy(stgs[b], accum.at[dstb.at[g]], add=True)
                d = dn

    @pl.when(c == 0)
    def _():
        pipeline(hw0)

    @pl.when(c == 1)
    def _():
        pipeline(hw1)

    plsc.subcore_barrier()
    pltpu.sync_copy(accum.at[pl.ds(s * 640, 640)],
                    agg.at[c, pl.ds(s * 640, 640)])


def _sc_agg(zhbm, hw0, hw1, srcr, dstr):
    return pl.kernel(
        _sc_agg_body,
        out_type=[jax.ShapeDtypeStruct((NC, NP, HH), f32)],
        mesh=_mesh(),
        scratch_types=[
            pltpu.VMEM((NGRP, 128), i32),
            pltpu.VMEM((NGRP, 128), i32),
            pltpu.VMEM((128, HH), f32),
            pltpu.VMEM((128, HH), f32),
            pltpu.VMEM_SHARED((NP, HH), f32),
            pltpu.SemaphoreType.DMA,
            pltpu.SemaphoreType.DMA,
        ],
    )(zhbm, hw0, hw1, srcr, dstr)[0]


def _tc0_body(degw, ag, x0, dep, temb, demb, w0,
              h_out, dinv_out, hw0_out, hw1_out):
    oh_t = (lax.broadcasted_iota(i32, (BR, 98), 1) == x0[...]).astype(f32)
    oh_d = (lax.broadcasted_iota(i32, (BR, MAXD + 1), 1)
            == dep[...]).astype(f32)
    h = (ag[...] + jnp.dot(oh_t, temb[...], preferred_element_type=f32)
         + jnp.dot(oh_d, demb[...], preferred_element_type=f32))
    deg = degw[0, :, 0] + degw[1, :, 0] + 1.0
    dinv = lax.rsqrt(jnp.maximum(deg, 1.0))[:, None]
    hw = jnp.dot(h, w0[...], preferred_element_type=f32) * dinv
    h_out[...] = h
    dinv_out[...] = dinv
    hw0_out[...] = hw[:, :HH]
    hw1_out[...] = hw[:, HH:]


def _tc0(degw, ag, x0, dep, temb, demb, w0):
    return pl.pallas_call(
        _tc0_body,
        grid=(NRB,),
        in_specs=[
            pl.BlockSpec((NC, BR, HH), lambda r: (0, r, 0)),
            pl.BlockSpec((BR, H), lambda r: (r, 0)),
            pl.BlockSpec((BR, 1), lambda r: (r, 0)),
            pl.BlockSpec((BR, 1), lambda r: (r, 0)),
            pl.BlockSpec((98, H), lambda r: (0, 0)),
            pl.BlockSpec((MAXD + 1, H), lambda r: (0, 0)),
            pl.BlockSpec((H, H), lambda r: (0, 0)),
        ],
        out_specs=[
            pl.BlockSpec((BR, H), lambda r: (r, 0)),
            pl.BlockSpec((BR, 1), lambda r: (r, 0)),
            pl.BlockSpec((BR, HH), lambda r: (r, 0)),
            pl.BlockSpec((BR, HH), lambda r: (r, 0)),
        ],
        out_shape=[
            jax.ShapeDtypeStruct((NP, H), f32),
            jax.ShapeDtypeStruct((NP, 1), f32),
            jax.ShapeDtypeStruct((NP, HH), f32),
            jax.ShapeDtypeStruct((NP, HH), f32),
        ],
    )(degw, ag, x0, dep, temb, demb, w0)


def _tc_stats_body(agg, hwa, hwb, dinv, bvec, conv_out, psum, psumsq):
    r = pl.program_id(0)
    a = jnp.concatenate([agg[0], agg[1]], axis=1)
    hwp = jnp.concatenate([hwa[...], hwb[...]], axis=1)
    conv = (a + hwp) * dinv[...] + bvec[...]
    row = lax.broadcasted_iota(i32, (BR, 1), 0) + r * BR
    mask = (row < N).astype(f32)
    cm = conv * mask
    conv_out[...] = conv
    psum[0, 0] = jnp.sum(cm, axis=0)
    psumsq[0, 0] = jnp.sum(cm * cm, axis=0)


def _tc_stats(agg, hw0, hw1, dinv, bvec):
    return pl.pallas_call(
        _tc_stats_body,
        grid=(NRB,),
        in_specs=[
            pl.BlockSpec((NC, BR, HH), lambda r: (0, r, 0)),
            pl.BlockSpec((BR, HH), lambda r: (r, 0)),
            pl.BlockSpec((BR, HH), lambda r: (r, 0)),
            pl.BlockSpec((BR, 1), lambda r: (r, 0)),
            pl.BlockSpec((1, H), lambda r: (0, 0)),
        ],
        out_specs=[
            pl.BlockSpec((BR, H), lambda r: (r, 0)),
            pl.BlockSpec((1, 1, H), lambda r: (r, 0, 0)),
            pl.BlockSpec((1, 1, H), lambda r: (r, 0, 0)),
        ],
        out_shape=[
            jax.ShapeDtypeStruct((NP, H), f32),
            jax.ShapeDtypeStruct((NRB, 1, H), f32),
            jax.ShapeDtypeStruct((NRB, 1, H), f32),
        ],
    )(agg, hw0, hw1, dinv, bvec)


def _tc_norm_mid_body(conv, psum, psumsq, hid, dinv, gam, bet, wn,
                      h_out, hw0_out, hw1_out):
    mu = jnp.sum(psum[...], axis=0) / N
    var = jnp.sum(psumsq[...], axis=0) / N - mu * mu
    hn = (conv[...] - mu) * lax.rsqrt(var + 1e-5) * gam[...] + bet[...]
    hn = jnp.maximum(hn, 0.0) + hid[...]
    h_out[...] = hn
    nhw = jnp.dot(hn, wn[...], preferred_element_type=f32) * dinv[...]
    hw0_out[...] = nhw[:, :HH]
    hw1_out[...] = nhw[:, HH:]


def _tc_norm_mid(conv, psum, psumsq, hid, dinv, gam, bet, wn):
    return pl.pallas_call(
        _tc_norm_mid_body,
        grid=(NRB,),
        in_specs=[
            pl.BlockSpec((BR, H), lambda r: (r, 0)),
            pl.BlockSpec((NRB, 1, H), lambda r: (0, 0, 0)),
            pl.BlockSpec((NRB, 1, H), lambda r: (0, 0, 0)),
            pl.BlockSpec((BR, H), lambda r: (r, 0)),
            pl.BlockSpec((BR, 1), lambda r: (r, 0)),
            pl.BlockSpec((1, H), lambda r: (0, 0)),
            pl.BlockSpec((1, H), lambda r: (0, 0)),
            pl.BlockSpec((H, H), lambda r: (0, 0)),
        ],
        out_specs=[
            pl.BlockSpec((BR, H), lambda r: (r, 0)),
            pl.BlockSpec((BR, HH), lambda r: (r, 0)),
            pl.BlockSpec((BR, HH), lambda r: (r, 0)),
        ],
        out_shape=[
            jax.ShapeDtypeStruct((NP, H), f32),
            jax.ShapeDtypeStruct((NP, HH), f32),
            jax.ShapeDtypeStruct((NP, HH), f32),
        ],
    )(conv, psum, psumsq, hid, dinv, gam, bet, wn)


def _tc_norm_last_body(conv, psum, psumsq, hid, gam, bet, h_out):
    mu = jnp.sum(psum[...], axis=0) / N
    var = jnp.sum(psumsq[...], axis=0) / N - mu * mu
    hn = (conv[...] - mu) * lax.rsqrt(var + 1e-5) * gam[...] + bet[...]
    h_out[...] = jnp.maximum(hn, 0.0) + hid[...]


def _tc_norm_last(conv, psum, psumsq, hid, gam, bet):
    return pl.pallas_call(
        _tc_norm_last_body,
        grid=(NRB,),
        in_specs=[
            pl.BlockSpec((BR, H), lambda r: (r, 0)),
            pl.BlockSpec((NRB, 1, H), lambda r: (0, 0, 0)),
            pl.BlockSpec((NRB, 1, H), lambda r: (0, 0, 0)),
            pl.BlockSpec((BR, H), lambda r: (r, 0)),
            pl.BlockSpec((1, H), lambda r: (0, 0)),
            pl.BlockSpec((1, H), lambda r: (0, 0)),
        ],
        out_specs=pl.BlockSpec((BR, H), lambda r: (r, 0)),
        out_shape=jax.ShapeDtypeStruct((NP, H), f32),
    )(conv, psum, psumsq, hid, gam, bet)


def _tc_pool_body(h, batch, wt, bt, out):
    oh = (lax.broadcasted_iota(i32, (G, NP), 0) == batch[...]).astype(f32)
    cnt = jnp.sum(oh, axis=1, keepdims=True)
    pooled = jnp.dot(oh, h[...], preferred_element_type=f32)
    pooled = pooled / jnp.maximum(cnt, 1.0)
    out[0] = jnp.dot(pooled, wt[0], preferred_element_type=f32) + bt[0]


def _tc_pool(h, batch2d, wt, bt):
    return pl.pallas_call(
        _tc_pool_body,
        grid=(5,),
        in_specs=[
            pl.BlockSpec((NP, H), lambda s: (0, 0)),
            pl.BlockSpec((1, NP), lambda s: (0, 0)),
            pl.BlockSpec((1, H, VP), lambda s: (s, 0, 0)),
            pl.BlockSpec((1, 1, VP), lambda s: (s, 0, 0)),
        ],
        out_specs=pl.BlockSpec((1, G, VP), lambda s: (s, 0, 0)),
        out_shape=jax.ShapeDtypeStruct((5, G, VP), f32),
    )(h, batch2d, wt, bt)


def kernel(x, edge_index, node_depth, batch, type_emb, attr_emb, depth_emb,
           Ws, bs, gammas, betas, Wt, bt):
    x0 = jnp.pad(x[:, 0], (0, NP - N))
    x1 = jnp.pad(x[:, 1], (0, NP - N))
    dep = jnp.pad(jnp.minimum(node_depth[:, 0], MAXD), (0, NP - N))
    pad_e = jnp.full((EP - E,), N, i32)
    src_p = jnp.concatenate([edge_index[0], pad_e])
    dst_p = jnp.concatenate([edge_index[1], pad_e])
    srcr = src_p.reshape(NS, NCHUNK, 128)
    dstr = dst_p.reshape(NS, NCHUNK, 128)
    dstr32 = dst_p.reshape(32, 40, 128)
    zconst = jnp.zeros((128, HH), f32)
    oconst = jnp.ones((128, HH), f32)

    ag = _sc_embed(x1, attr_emb)
    degw = _sc_deg(zconst, oconst, dstr32)
    h, dinv, hw0, hw1 = _tc0(degw, ag, x0.reshape(NP, 1),
                             dep.reshape(NP, 1), type_emb, depth_emb,
                             Ws[0])
    for l in range(L):
        agg = _sc_agg(zconst, hw0, hw1, srcr, dstr)
        gam = gammas[l].reshape(1, H)
        bet = betas[l].reshape(1, H)
        bv = bs[l].reshape(1, H)
        conv, psum, psumsq = _tc_stats(agg, hw0, hw1, dinv, bv)
        if l < L - 1:
            h, hw0, hw1 = _tc_norm_mid(conv, psum, psumsq, h, dinv, gam,
                                       bet, Ws[l + 1])
        else:
            h = _tc_norm_last(conv, psum, psumsq, h, gam, bet)

    batch_p = jnp.pad(batch, (0, NP - N), constant_values=2 * G)
    wt_p = jnp.pad(Wt, ((0, 0), (0, 0), (0, VP - V2)))
    bt_p = jnp.pad(bt, ((0, 0), (0, VP - V2))).reshape(5, 1, VP)
    preds = _tc_pool(h, batch_p.reshape(1, NP), wt_p, bt_p)
    return preds[:, :, :V2]
